# Initial kernel scaffold; baseline (speedup 1.0000x reference)
import jax
import jax.numpy as jnp
from jax.experimental import pallas as pl


def kernel(x, edge_index, attention, Wcz, bcz, Wcr, bcr, Wch, bch, Wlz, blz, Wlr, blr, Wlh, blh, W1, b1, Wout, bout):
    raise NotImplementedError("write your pallas kernel here")


# speedup vs baseline: 43.9606x; 43.9606x over previous
import functools

import jax
import jax.numpy as jnp
from jax import lax
from jax.experimental import pallas as pl
from jax.experimental.pallas import tpu as pltpu
from jax.experimental.pallas import tpu_sc as plsc

N = 10000
NPAD = 10240
E = 320000
F_IN = 128
HID = 32
FW = 2 * HID
PERIODS = 12
NC = 2
NS = 16
EB = 125
GQ = 4
NBLK = 10
BN = N // NBLK

ROWS_PER_TILE = NPAD // NS
ZR = 80
NBD = E // (NC * NS * EB)
NBE = E // (NS * EB)

_SC_PARAMS = pltpu.CompilerParams(use_tc_tiling_on_sc=False)


def _sc_mesh():
    return plsc.VectorSubcoreMesh(
        core_axis_name="c", subcore_axis_name="s", num_cores=NC, num_subcores=NS
    )


def _zero_vmem_rows(ref, rows, width):
    @pl.loop(0, rows)
    def _(i):
        for q in range(width // 16):
            ref[i, pl.ds(q * 16, 16)] = jnp.zeros((16,), jnp.float32)


def _sc_degree(dst_r):
    @functools.partial(
        pl.kernel,
        out_type=jax.ShapeDtypeStruct((NC, NPAD, 16), jnp.float32),
        mesh=_sc_mesh(),
        compiler_params=_SC_PARAMS,
        scratch_types=[
            pltpu.VMEM((NBD, EB), jnp.int32),
            pltpu.VMEM((EB, 16), jnp.float32),
            pltpu.VMEM((ZR, 16), jnp.float32),
            pltpu.VMEM_SHARED((NPAD, 16), jnp.float32),
        ],
    )
    def k(dst_hbm, out_hbm, idx_v, ones_v, zer_v, acc_sh):
        c = lax.axis_index("c")
        s = lax.axis_index("s")
        wid = c * NS + s
        pltpu.sync_copy(dst_hbm.at[wid], idx_v)

        @pl.loop(0, EB)
        def _(i):
            ones_v[i, :] = jnp.ones((16,), jnp.float32)

        _zero_vmem_rows(zer_v, ZR, 16)
        row0 = s * ROWS_PER_TILE
        for z in range(ROWS_PER_TILE // ZR):
            pltpu.sync_copy(zer_v, acc_sh.at[pl.ds(row0 + z * ZR, ZR)])
        plsc.subcore_barrier()

        @pl.loop(0, NBD)
        def _(j):
            pltpu.sync_copy(ones_v, acc_sh.at[idx_v.at[j]], add=True)

        plsc.subcore_barrier()
        pltpu.sync_copy(
            acc_sh.at[pl.ds(row0, ROWS_PER_TILE)],
            out_hbm.at[c, pl.ds(row0, ROWS_PER_TILE)],
        )

    return k(dst_r)


def _sc_aggregate(src_r, dst_r, mprime):
    pp = PERIODS // NC
    ngroups = NBE // GQ

    @functools.partial(
        pl.kernel,
        out_type=jax.ShapeDtypeStruct((PERIODS, NPAD, FW), jnp.float32),
        mesh=_sc_mesh(),
        compiler_params=_SC_PARAMS,
        scratch_types=[
            pltpu.VMEM((NBE, EB), jnp.int32),
            pltpu.VMEM((NBE, EB), jnp.int32),
            pltpu.VMEM((GQ, EB, FW), jnp.float32),
            pltpu.VMEM((ZR, FW), jnp.float32),
            pltpu.VMEM_SHARED((NPAD, FW), jnp.float32),
            pltpu.SemaphoreType.DMA((GQ,)),
        ],
    )
    def k(src_hbm, dst_hbm, mp_hbm, out_hbm, sidx, didx, gbuf, zer_v, acc_sh, gsem):
        c = lax.axis_index("c")
        s = lax.axis_index("s")
        pltpu.sync_copy(src_hbm.at[s], sidx)
        pltpu.sync_copy(dst_hbm.at[s], didx)
        _zero_vmem_rows(zer_v, ZR, FW)

        row0 = s * ROWS_PER_TILE
        for p_i in range(pp):
            p = c * pp + p_i
            tab = mp_hbm.at[p]
            for z in range(ROWS_PER_TILE // ZR):
                pltpu.sync_copy(zer_v, acc_sh.at[pl.ds(row0 + z * ZR, ZR)])
            plsc.subcore_barrier()

            @pl.loop(0, ngroups)
            def _(g):
                base = g * GQ
                cps = []
                for b in range(GQ):
                    cps.append(
                        pltpu.async_copy(
                            tab.at[sidx.at[base + b]], gbuf.at[b], gsem.at[b]
                        )
                    )
                for b in range(GQ):
                    cps[b].wait()
                    pltpu.sync_copy(gbuf.at[b], acc_sh.at[didx.at[base + b]], add=True)

            plsc.subcore_barrier()
            pltpu.sync_copy(
                acc_sh.at[pl.ds(row0, ROWS_PER_TILE)],
                out_hbm.at[p, pl.ds(row0, ROWS_PER_TILE)],
            )
            plsc.subcore_barrier()

    return k(src_r, dst_r, mprime)


def _tc_prep(x_t, wcat, degpart):
    def body(x_ref, w_ref, d_ref, o_ref):
        deg = d_ref[0, :, :1] + d_ref[1, :, :1] + 1.0
        dinv = lax.rsqrt(deg)
        m = jnp.dot(x_ref[0], w_ref[...], preferred_element_type=jnp.float32)
        o_ref[0] = m * dinv

    return pl.pallas_call(
        body,
        grid=(PERIODS, NBLK),
        in_specs=[
            pl.BlockSpec((1, BN, F_IN), lambda p, b: (p, b, 0)),
            pl.BlockSpec((F_IN, FW), lambda p, b: (0, 0)),
            pl.BlockSpec((NC, BN, 16), lambda p, b: (0, b, 0)),
        ],
        out_specs=pl.BlockSpec((1, BN, FW), lambda p, b: (p, b, 0)),
        out_shape=jax.ShapeDtypeStruct((PERIODS, NPAD, FW), jnp.float32),
    )(x_t, wcat, degpart)


def _tc_final(agg, mprime, degpart, probs, bz, bh, w1, b1, wout, bout):
    def body(a_ref, m_ref, d_ref, pr_ref, bz_ref, bh_ref, w1_ref, b1_ref, wo_ref, bo_ref, o_ref):
        deg = d_ref[0, :, :1] + d_ref[1, :, :1] + 1.0
        dinv = lax.rsqrt(deg)
        acc = jnp.zeros((BN, HID), jnp.float32)
        for p in range(PERIODS):
            g = (a_ref[p] + m_ref[p]) * dinv
            zl = g[:, :HID] + bz_ref[...]
            hl = g[:, HID:] + bh_ref[...]
            hp = (1.0 - jax.nn.sigmoid(zl)) * jnp.tanh(hl)
            acc = acc + pr_ref[0, p] * hp
        h = jnp.maximum(acc, 0.0)
        h = jnp.maximum(
            jnp.dot(h, w1_ref[...], preferred_element_type=jnp.float32) + b1_ref[...],
            0.0,
        )
        o_ref[...] = (
            jnp.dot(h, wo_ref[...], preferred_element_type=jnp.float32) + bo_ref[...]
        )

    return pl.pallas_call(
        body,
        grid=(NBLK,),
        in_specs=[
            pl.BlockSpec((PERIODS, BN, FW), lambda b: (0, b, 0)),
            pl.BlockSpec((PERIODS, BN, FW), lambda b: (0, b, 0)),
            pl.BlockSpec((NC, BN, 16), lambda b: (0, b, 0)),
            pl.BlockSpec(memory_space=pltpu.SMEM),
            pl.BlockSpec((1, HID), lambda b: (0, 0)),
            pl.BlockSpec((1, HID), lambda b: (0, 0)),
            pl.BlockSpec((HID, HID), lambda b: (0, 0)),
            pl.BlockSpec((1, HID), lambda b: (0, 0)),
            pl.BlockSpec((HID, PERIODS), lambda b: (0, 0)),
            pl.BlockSpec((1, PERIODS), lambda b: (0, 0)),
        ],
        out_specs=pl.BlockSpec((BN, PERIODS), lambda b: (b, 0)),
        out_shape=jax.ShapeDtypeStruct((N, PERIODS), jnp.float32),
    )(agg, mprime, degpart, probs, bz, bh, w1, b1, wout, bout)


def kernel(x, edge_index, attention, Wcz, bcz, Wcr, bcr, Wch, bch, Wlz, blz, Wlr, blr, Wlh, blh, W1, b1, Wout, bout):
    src = edge_index[0]
    dst = edge_index[1]
    probs = jax.nn.softmax(attention).reshape(1, PERIODS)
    wcat = jnp.concatenate([Wcz @ Wlz[:HID], Wch @ Wlh[:HID]], axis=1)
    bz = (bcz @ Wlz[:HID] + blz).reshape(1, HID)
    bh = (bch @ Wlh[:HID] + blh).reshape(1, HID)
    x_t = jnp.transpose(x, (2, 0, 1))

    degpart = _sc_degree(dst.reshape(NC * NS, NBD, EB))
    mprime = _tc_prep(x_t, wcat, degpart)
    agg = _sc_aggregate(
        src.reshape(NS, NBE, EB), dst.reshape(NS, NBE, EB), mprime
    )
    return _tc_final(agg, mprime, degpart, probs, bz, bh, W1,
                     b1.reshape(1, HID), Wout, bout.reshape(1, PERIODS))


# speedup vs baseline: 54.7655x; 1.2458x over previous
import functools

import jax
import jax.numpy as jnp
from jax import lax
from jax.experimental import pallas as pl
from jax.experimental.pallas import tpu as pltpu
from jax.experimental.pallas import tpu_sc as plsc

N = 10000
NPAD = 10240
E = 320000
F_IN = 128
HID = 32
FW = 2 * HID
PERIODS = 12
NC = 2
NS = 16
EB = 125
GQ = 4
NBLK = 10
BN = N // NBLK

ROWS_PER_TILE = NPAD // NS
ZR = 80
NBD = E // (NC * NS * EB)
NBE = E // (NS * EB)

_SC_PARAMS = pltpu.CompilerParams(use_tc_tiling_on_sc=False)


def _sc_mesh():
    return plsc.VectorSubcoreMesh(
        core_axis_name="c", subcore_axis_name="s", num_cores=NC, num_subcores=NS
    )


def _zero_vmem_rows(ref, rows, width):
    @pl.loop(0, rows)
    def _(i):
        for q in range(width // 16):
            ref[i, pl.ds(q * 16, 16)] = jnp.zeros((16,), jnp.float32)


def _sc_degree(dst_r):
    @functools.partial(
        pl.kernel,
        out_type=jax.ShapeDtypeStruct((NC, NPAD, 16), jnp.float32),
        mesh=_sc_mesh(),
        compiler_params=_SC_PARAMS,
        scratch_types=[
            pltpu.VMEM((NBD, EB), jnp.int32),
            pltpu.VMEM((EB, 16), jnp.float32),
            pltpu.VMEM((ZR, 16), jnp.float32),
            pltpu.VMEM_SHARED((NPAD, 16), jnp.float32),
        ],
    )
    def k(dst_hbm, out_hbm, idx_v, ones_v, zer_v, acc_sh):
        c = lax.axis_index("c")
        s = lax.axis_index("s")
        wid = c * NS + s
        pltpu.sync_copy(dst_hbm.at[wid], idx_v)

        @pl.loop(0, EB)
        def _(i):
            ones_v[i, :] = jnp.ones((16,), jnp.float32)

        _zero_vmem_rows(zer_v, ZR, 16)
        row0 = s * ROWS_PER_TILE
        for z in range(ROWS_PER_TILE // ZR):
            pltpu.sync_copy(zer_v, acc_sh.at[pl.ds(row0 + z * ZR, ZR)])
        plsc.subcore_barrier()

        @pl.loop(0, NBD)
        def _(j):
            pltpu.sync_copy(ones_v, acc_sh.at[idx_v.at[j]], add=True)

        plsc.subcore_barrier()
        pltpu.sync_copy(
            acc_sh.at[pl.ds(row0, ROWS_PER_TILE)],
            out_hbm.at[c, pl.ds(row0, ROWS_PER_TILE)],
        )

    return k(dst_r)


def _sc_aggregate(src_r, dst_r, mprime):
    pp = PERIODS // NC
    ngroups = NBE // GQ

    @functools.partial(
        pl.kernel,
        out_type=jax.ShapeDtypeStruct((PERIODS, NPAD, FW), jnp.float32),
        mesh=_sc_mesh(),
        compiler_params=_SC_PARAMS,
        scratch_types=[
            pltpu.VMEM((NBE, EB), jnp.int32),
            pltpu.VMEM((NBE, EB), jnp.int32),
            pltpu.VMEM((GQ, EB, FW), jnp.float32),
            pltpu.VMEM((ZR, FW), jnp.float32),
            pltpu.VMEM_SHARED((NPAD, FW), jnp.float32),
            pltpu.SemaphoreType.DMA((GQ,)),
            pltpu.SemaphoreType.DMA((GQ,)),
        ],
    )
    def k(src_hbm, dst_hbm, mp_hbm, out_hbm, sidx, didx, gbuf, zer_v, acc_sh, gsem, ssem):
        c = lax.axis_index("c")
        s = lax.axis_index("s")
        pltpu.sync_copy(src_hbm.at[s], sidx)
        pltpu.sync_copy(dst_hbm.at[s], didx)
        _zero_vmem_rows(zer_v, ZR, FW)

        row0 = s * ROWS_PER_TILE
        for p_i in range(pp):
            p = c * pp + p_i
            tab = mp_hbm.at[p]
            for z in range(ROWS_PER_TILE // ZR):
                pltpu.sync_copy(zer_v, acc_sh.at[pl.ds(row0 + z * ZR, ZR)])
            plsc.subcore_barrier()

            for b in range(GQ):
                pltpu.async_copy(tab.at[sidx.at[b]], gbuf.at[b], gsem.at[b])

            @pl.loop(0, ngroups)
            def _(g):
                base = g * GQ
                for b in range(GQ):
                    pltpu.make_async_copy(
                        tab.at[sidx.at[base + b]], gbuf.at[b], gsem.at[b]
                    ).wait()
                    pltpu.async_copy(
                        gbuf.at[b], acc_sh.at[didx.at[base + b]], ssem.at[b],
                        add=True,
                    )

                @pl.when(g < ngroups - 1)
                def _():
                    nbase = base + GQ
                    for b in range(GQ):
                        pltpu.make_async_copy(
                            gbuf.at[b], acc_sh.at[didx.at[base + b]], ssem.at[b]
                        ).wait()
                        pltpu.async_copy(
                            tab.at[sidx.at[nbase + b]], gbuf.at[b], gsem.at[b]
                        )

            for b in range(GQ):
                pltpu.make_async_copy(
                    gbuf.at[b], acc_sh.at[didx.at[b]], ssem.at[b]
                ).wait()
            plsc.subcore_barrier()
            pltpu.sync_copy(
                acc_sh.at[pl.ds(row0, ROWS_PER_TILE)],
                out_hbm.at[p, pl.ds(row0, ROWS_PER_TILE)],
            )
            plsc.subcore_barrier()

    return k(src_r, dst_r, mprime)


def _tc_prep(x_t, wcat, degpart):
    def body(x_ref, w_ref, d_ref, o_ref):
        deg = d_ref[0, :, :1] + d_ref[1, :, :1] + 1.0
        dinv = lax.rsqrt(deg)
        m = jnp.dot(x_ref[0], w_ref[...], preferred_element_type=jnp.float32)
        o_ref[0] = m * dinv

    return pl.pallas_call(
        body,
        grid=(PERIODS, NBLK),
        in_specs=[
            pl.BlockSpec((1, BN, F_IN), lambda p, b: (p, b, 0)),
            pl.BlockSpec((F_IN, FW), lambda p, b: (0, 0)),
            pl.BlockSpec((NC, BN, 16), lambda p, b: (0, b, 0)),
        ],
        out_specs=pl.BlockSpec((1, BN, FW), lambda p, b: (p, b, 0)),
        out_shape=jax.ShapeDtypeStruct((PERIODS, NPAD, FW), jnp.float32),
    )(x_t, wcat, degpart)


def _tc_final(agg, mprime, degpart, probs, bz, bh, w1, b1, wout, bout):
    def body(a_ref, m_ref, d_ref, pr_ref, bz_ref, bh_ref, w1_ref, b1_ref, wo_ref, bo_ref, o_ref):
        deg = d_ref[0, :, :1] + d_ref[1, :, :1] + 1.0
        dinv = lax.rsqrt(deg)
        acc = jnp.zeros((BN, HID), jnp.float32)
        for p in range(PERIODS):
            g = (a_ref[p] + m_ref[p]) * dinv
            zl = g[:, :HID] + bz_ref[...]
            hl = g[:, HID:] + bh_ref[...]
            hp = (1.0 - jax.nn.sigmoid(zl)) * jnp.tanh(hl)
            acc = acc + pr_ref[0, p] * hp
        h = jnp.maximum(acc, 0.0)
        h = jnp.maximum(
            jnp.dot(h, w1_ref[...], preferred_element_type=jnp.float32) + b1_ref[...],
            0.0,
        )
        o_ref[...] = (
            jnp.dot(h, wo_ref[...], preferred_element_type=jnp.float32) + bo_ref[...]
        )

    return pl.pallas_call(
        body,
        grid=(NBLK,),
        in_specs=[
            pl.BlockSpec((PERIODS, BN, FW), lambda b: (0, b, 0)),
            pl.BlockSpec((PERIODS, BN, FW), lambda b: (0, b, 0)),
            pl.BlockSpec((NC, BN, 16), lambda b: (0, b, 0)),
            pl.BlockSpec(memory_space=pltpu.SMEM),
            pl.BlockSpec((1, HID), lambda b: (0, 0)),
            pl.BlockSpec((1, HID), lambda b: (0, 0)),
            pl.BlockSpec((HID, HID), lambda b: (0, 0)),
            pl.BlockSpec((1, HID), lambda b: (0, 0)),
            pl.BlockSpec((HID, PERIODS), lambda b: (0, 0)),
            pl.BlockSpec((1, PERIODS), lambda b: (0, 0)),
        ],
        out_specs=pl.BlockSpec((BN, PERIODS), lambda b: (b, 0)),
        out_shape=jax.ShapeDtypeStruct((N, PERIODS), jnp.float32),
    )(agg, mprime, degpart, probs, bz, bh, w1, b1, wout, bout)


def kernel(x, edge_index, attention, Wcz, bcz, Wcr, bcr, Wch, bch, Wlz, blz, Wlr, blr, Wlh, blh, W1, b1, Wout, bout):
    src = edge_index[0]
    dst = edge_index[1]
    probs = jax.nn.softmax(attention).reshape(1, PERIODS)
    wcat = jnp.concatenate([Wcz @ Wlz[:HID], Wch @ Wlh[:HID]], axis=1)
    bz = (bcz @ Wlz[:HID] + blz).reshape(1, HID)
    bh = (bch @ Wlh[:HID] + blh).reshape(1, HID)
    x_t = jnp.transpose(x, (2, 0, 1))

    degpart = _sc_degree(dst.reshape(NC * NS, NBD, EB))
    mprime = _tc_prep(x_t, wcat, degpart)
    agg = _sc_aggregate(
        src.reshape(NS, NBE, EB), dst.reshape(NS, NBE, EB), mprime
    )
    return _tc_final(agg, mprime, degpart, probs, bz, bh, W1,
                     b1.reshape(1, HID), Wout, bout.reshape(1, PERIODS))


# speedup vs baseline: 54.8796x; 1.0021x over previous
import functools

import jax
import jax.numpy as jnp
from jax import lax
from jax.experimental import pallas as pl
from jax.experimental.pallas import tpu as pltpu
from jax.experimental.pallas import tpu_sc as plsc

N = 10000
NPAD = 10240
E = 320000
F_IN = 128
HID = 32
FW = 2 * HID
PERIODS = 12
NC = 2
NS = 16
EB = 125
GQ = 4
NBLK = 10
BN = N // NBLK

ROWS_PER_TILE = NPAD // NS
ZR = 80
NBD = E // (NC * NS * EB)
NBE = E // (NS * EB)

_SC_PARAMS = pltpu.CompilerParams(use_tc_tiling_on_sc=False)


def _sc_mesh():
    return plsc.VectorSubcoreMesh(
        core_axis_name="c", subcore_axis_name="s", num_cores=NC, num_subcores=NS
    )


def _zero_vmem_rows(ref, rows, width):
    @pl.loop(0, rows)
    def _(i):
        for q in range(width // 16):
            ref[i, pl.ds(q * 16, 16)] = jnp.zeros((16,), jnp.float32)


def _sc_degree(dst_r):
    @functools.partial(
        pl.kernel,
        out_type=jax.ShapeDtypeStruct((NC, NPAD, 16), jnp.float32),
        mesh=_sc_mesh(),
        compiler_params=_SC_PARAMS,
        scratch_types=[
            pltpu.VMEM((NBD, EB), jnp.int32),
            pltpu.VMEM((EB, 16), jnp.float32),
            pltpu.VMEM((ZR, 16), jnp.float32),
            pltpu.VMEM_SHARED((NPAD, 16), jnp.float32),
        ],
    )
    def k(dst_hbm, out_hbm, idx_v, ones_v, zer_v, acc_sh):
        c = lax.axis_index("c")
        s = lax.axis_index("s")
        wid = c * NS + s
        pltpu.sync_copy(dst_hbm.at[wid], idx_v)

        @pl.loop(0, EB)
        def _(i):
            ones_v[i, :] = jnp.ones((16,), jnp.float32)

        _zero_vmem_rows(zer_v, ZR, 16)
        row0 = s * ROWS_PER_TILE
        for z in range(ROWS_PER_TILE // ZR):
            pltpu.sync_copy(zer_v, acc_sh.at[pl.ds(row0 + z * ZR, ZR)])
        plsc.subcore_barrier()

        @pl.loop(0, NBD)
        def _(j):
            pltpu.sync_copy(ones_v, acc_sh.at[idx_v.at[j]], add=True)

        plsc.subcore_barrier()
        pltpu.sync_copy(
            acc_sh.at[pl.ds(row0, ROWS_PER_TILE)],
            out_hbm.at[c, pl.ds(row0, ROWS_PER_TILE)],
        )

    return k(dst_r)


def _sc_aggregate(src_r, dst_r, mprime):
    pp = PERIODS // NC
    ngroups = NBE // GQ

    @functools.partial(
        pl.kernel,
        out_type=jax.ShapeDtypeStruct((PERIODS, NPAD, FW), jnp.float32),
        mesh=_sc_mesh(),
        compiler_params=_SC_PARAMS,
        scratch_types=[
            pltpu.VMEM((NBE, EB), jnp.int32),
            pltpu.VMEM((NBE, EB), jnp.int32),
            pltpu.VMEM((GQ, EB, FW), jnp.float32),
            pltpu.VMEM_SHARED((NPAD, FW), jnp.float32),
            pltpu.SemaphoreType.DMA((GQ,)),
            pltpu.SemaphoreType.DMA((GQ,)),
        ],
    )
    def k(src_hbm, dst_hbm, mp_hbm, out_hbm, sidx, didx, gbuf, acc_sh, gsem, ssem):
        c = lax.axis_index("c")
        s = lax.axis_index("s")
        pltpu.sync_copy(src_hbm.at[s], sidx)
        pltpu.sync_copy(dst_hbm.at[s], didx)

        row0 = s * ROWS_PER_TILE
        for p_i in range(pp):
            p = c * pp + p_i
            tab = mp_hbm.at[p]
            pltpu.sync_copy(
                tab.at[pl.ds(row0, ROWS_PER_TILE)],
                acc_sh.at[pl.ds(row0, ROWS_PER_TILE)],
            )
            plsc.subcore_barrier()

            for b in range(GQ):
                pltpu.async_copy(tab.at[sidx.at[b]], gbuf.at[b], gsem.at[b])

            @pl.loop(0, ngroups)
            def _(g):
                base = g * GQ
                for b in range(GQ):
                    pltpu.make_async_copy(
                        tab.at[sidx.at[base + b]], gbuf.at[b], gsem.at[b]
                    ).wait()
                    pltpu.async_copy(
                        gbuf.at[b], acc_sh.at[didx.at[base + b]], ssem.at[b],
                        add=True,
                    )

                @pl.when(g < ngroups - 1)
                def _():
                    nbase = base + GQ
                    for b in range(GQ):
                        pltpu.make_async_copy(
                            gbuf.at[b], acc_sh.at[didx.at[base + b]], ssem.at[b]
                        ).wait()
                        pltpu.async_copy(
                            tab.at[sidx.at[nbase + b]], gbuf.at[b], gsem.at[b]
                        )

            for b in range(GQ):
                pltpu.make_async_copy(
                    gbuf.at[b], acc_sh.at[didx.at[b]], ssem.at[b]
                ).wait()
            plsc.subcore_barrier()
            pltpu.sync_copy(
                acc_sh.at[pl.ds(row0, ROWS_PER_TILE)],
                out_hbm.at[p, pl.ds(row0, ROWS_PER_TILE)],
            )
            plsc.subcore_barrier()

    return k(src_r, dst_r, mprime)


def _tc_prep(x_t, wcat, degpart):
    def body(x_ref, w_ref, d_ref, o_ref):
        deg = d_ref[0, :, :1] + d_ref[1, :, :1] + 1.0
        dinv = lax.rsqrt(deg)
        m = jnp.dot(x_ref[0], w_ref[...], preferred_element_type=jnp.float32)
        o_ref[0] = m * dinv

    return pl.pallas_call(
        body,
        grid=(PERIODS, NBLK),
        in_specs=[
            pl.BlockSpec((1, BN, F_IN), lambda p, b: (p, b, 0)),
            pl.BlockSpec((F_IN, FW), lambda p, b: (0, 0)),
            pl.BlockSpec((NC, BN, 16), lambda p, b: (0, b, 0)),
        ],
        out_specs=pl.BlockSpec((1, BN, FW), lambda p, b: (p, b, 0)),
        out_shape=jax.ShapeDtypeStruct((PERIODS, NPAD, FW), jnp.float32),
    )(x_t, wcat, degpart)


def _tc_final(agg, degpart, probs, bz, bh, w1, b1, wout, bout):
    def body(a_ref, d_ref, pr_ref, bz_ref, bh_ref, w1_ref, b1_ref, wo_ref, bo_ref, o_ref):
        deg = d_ref[0, :, :1] + d_ref[1, :, :1] + 1.0
        dinv = lax.rsqrt(deg)
        acc = jnp.zeros((BN, HID), jnp.float32)
        for p in range(PERIODS):
            g = a_ref[p] * dinv
            zl = g[:, :HID] + bz_ref[...]
            hl = g[:, HID:] + bh_ref[...]
            hp = (1.0 - jax.nn.sigmoid(zl)) * jnp.tanh(hl)
            acc = acc + pr_ref[0, p] * hp
        h = jnp.maximum(acc, 0.0)
        h = jnp.maximum(
            jnp.dot(h, w1_ref[...], preferred_element_type=jnp.float32) + b1_ref[...],
            0.0,
        )
        o_ref[...] = (
            jnp.dot(h, wo_ref[...], preferred_element_type=jnp.float32) + bo_ref[...]
        )

    return pl.pallas_call(
        body,
        grid=(NBLK,),
        in_specs=[
            pl.BlockSpec((PERIODS, BN, FW), lambda b: (0, b, 0)),
            pl.BlockSpec((NC, BN, 16), lambda b: (0, b, 0)),
            pl.BlockSpec(memory_space=pltpu.SMEM),
            pl.BlockSpec((1, HID), lambda b: (0, 0)),
            pl.BlockSpec((1, HID), lambda b: (0, 0)),
            pl.BlockSpec((HID, HID), lambda b: (0, 0)),
            pl.BlockSpec((1, HID), lambda b: (0, 0)),
            pl.BlockSpec((HID, PERIODS), lambda b: (0, 0)),
            pl.BlockSpec((1, PERIODS), lambda b: (0, 0)),
        ],
        out_specs=pl.BlockSpec((BN, PERIODS), lambda b: (b, 0)),
        out_shape=jax.ShapeDtypeStruct((N, PERIODS), jnp.float32),
    )(agg, degpart, probs, bz, bh, w1, b1, wout, bout)


def kernel(x, edge_index, attention, Wcz, bcz, Wcr, bcr, Wch, bch, Wlz, blz, Wlr, blr, Wlh, blh, W1, b1, Wout, bout):
    src = edge_index[0]
    dst = edge_index[1]
    probs = jax.nn.softmax(attention).reshape(1, PERIODS)
    wcat = jnp.concatenate([Wcz @ Wlz[:HID], Wch @ Wlh[:HID]], axis=1)
    bz = (bcz @ Wlz[:HID] + blz).reshape(1, HID)
    bh = (bch @ Wlh[:HID] + blh).reshape(1, HID)
    x_t = jnp.transpose(x, (2, 0, 1))

    degpart = _sc_degree(dst.reshape(NC * NS, NBD, EB))
    mprime = _tc_prep(x_t, wcat, degpart)
    agg = _sc_aggregate(
        src.reshape(NS, NBE, EB), dst.reshape(NS, NBE, EB), mprime
    )
    return _tc_final(agg, degpart, probs, bz, bh, W1,
                     b1.reshape(1, HID), Wout, bout.reshape(1, PERIODS))


# speedup vs baseline: 55.7430x; 1.0157x over previous
import functools

import jax
import jax.numpy as jnp
from jax import lax
from jax.experimental import pallas as pl
from jax.experimental.pallas import tpu as pltpu
from jax.experimental.pallas import tpu_sc as plsc

N = 10000
NPAD = 10240
E = 320000
F_IN = 128
HID = 32
FW = 2 * HID
PERIODS = 12
NC = 2
NS = 16
EB = 125
GQ = 5
NBLK = 10
BN = N // NBLK

ROWS_PER_TILE = NPAD // NS
ZR = 80
NBD = E // (NC * NS * EB)
NBE = E // (NS * EB)

_SC_PARAMS = pltpu.CompilerParams(use_tc_tiling_on_sc=False)


def _sc_mesh():
    return plsc.VectorSubcoreMesh(
        core_axis_name="c", subcore_axis_name="s", num_cores=NC, num_subcores=NS
    )


def _zero_vmem_rows(ref, rows, width):
    @pl.loop(0, rows)
    def _(i):
        for q in range(width // 16):
            ref[i, pl.ds(q * 16, 16)] = jnp.zeros((16,), jnp.float32)


def _sc_degree(dst_r):
    @functools.partial(
        pl.kernel,
        out_type=jax.ShapeDtypeStruct((NC, NPAD, 16), jnp.float32),
        mesh=_sc_mesh(),
        compiler_params=_SC_PARAMS,
        scratch_types=[
            pltpu.VMEM((NBD, EB), jnp.int32),
            pltpu.VMEM((EB, 16), jnp.float32),
            pltpu.VMEM((ZR, 16), jnp.float32),
            pltpu.VMEM_SHARED((NPAD, 16), jnp.float32),
        ],
    )
    def k(dst_hbm, out_hbm, idx_v, ones_v, zer_v, acc_sh):
        c = lax.axis_index("c")
        s = lax.axis_index("s")
        wid = c * NS + s
        pltpu.sync_copy(dst_hbm.at[wid], idx_v)

        @pl.loop(0, EB)
        def _(i):
            ones_v[i, :] = jnp.ones((16,), jnp.float32)

        _zero_vmem_rows(zer_v, ZR, 16)
        row0 = s * ROWS_PER_TILE
        for z in range(ROWS_PER_TILE // ZR):
            pltpu.sync_copy(zer_v, acc_sh.at[pl.ds(row0 + z * ZR, ZR)])
        plsc.subcore_barrier()

        @pl.loop(0, NBD)
        def _(j):
            pltpu.sync_copy(ones_v, acc_sh.at[idx_v.at[j]], add=True)

        plsc.subcore_barrier()
        pltpu.sync_copy(
            acc_sh.at[pl.ds(row0, ROWS_PER_TILE)],
            out_hbm.at[c, pl.ds(row0, ROWS_PER_TILE)],
        )

    return k(dst_r)


def _sc_aggregate(src_r, dst_r, mprime):
    pp = PERIODS // NC
    ngroups = NBE // GQ

    @functools.partial(
        pl.kernel,
        out_type=jax.ShapeDtypeStruct((PERIODS, NPAD, FW), jnp.float32),
        mesh=_sc_mesh(),
        compiler_params=_SC_PARAMS,
        scratch_types=[
            pltpu.VMEM((NBE, EB), jnp.int32),
            pltpu.VMEM((NBE, EB), jnp.int32),
            pltpu.VMEM((GQ, EB, FW), jnp.float32),
            pltpu.VMEM_SHARED((NPAD, FW), jnp.float32),
            pltpu.SemaphoreType.DMA((GQ,)),
            pltpu.SemaphoreType.DMA((GQ,)),
        ],
    )
    def k(src_hbm, dst_hbm, mp_hbm, out_hbm, sidx, didx, gbuf, acc_sh, gsem, ssem):
        c = lax.axis_index("c")
        s = lax.axis_index("s")
        pltpu.sync_copy(src_hbm.at[s], sidx)
        pltpu.sync_copy(dst_hbm.at[s], didx)

        row0 = s * ROWS_PER_TILE
        for p_i in range(pp):
            p = c * pp + p_i
            tab = mp_hbm.at[p]
            pltpu.sync_copy(
                tab.at[pl.ds(row0, ROWS_PER_TILE)],
                acc_sh.at[pl.ds(row0, ROWS_PER_TILE)],
            )
            plsc.subcore_barrier()

            for b in range(GQ):
                pltpu.async_copy(tab.at[sidx.at[b]], gbuf.at[b], gsem.at[b])

            @pl.loop(0, ngroups)
            def _(g):
                base = g * GQ
                for b in range(GQ):
                    pltpu.make_async_copy(
                        tab.at[sidx.at[base + b]], gbuf.at[b], gsem.at[b]
                    ).wait()
                    pltpu.async_copy(
                        gbuf.at[b], acc_sh.at[didx.at[base + b]], ssem.at[b],
                        add=True,
                    )

                @pl.when(g < ngroups - 1)
                def _():
                    nbase = base + GQ
                    for b in range(GQ):
                        pltpu.make_async_copy(
                            gbuf.at[b], acc_sh.at[didx.at[base + b]], ssem.at[b]
                        ).wait()
                        pltpu.async_copy(
                            tab.at[sidx.at[nbase + b]], gbuf.at[b], gsem.at[b]
                        )

            for b in range(GQ):
                pltpu.make_async_copy(
                    gbuf.at[b], acc_sh.at[didx.at[b]], ssem.at[b]
                ).wait()
            plsc.subcore_barrier()
            pltpu.sync_copy(
                acc_sh.at[pl.ds(row0, ROWS_PER_TILE)],
                out_hbm.at[p, pl.ds(row0, ROWS_PER_TILE)],
            )
            plsc.subcore_barrier()

    return k(src_r, dst_r, mprime)


def _tc_prep(x_t, wcat, degpart):
    def body(x_ref, w_ref, d_ref, o_ref):
        deg = d_ref[0, :, :1] + d_ref[1, :, :1] + 1.0
        dinv = lax.rsqrt(deg)
        m = jnp.dot(x_ref[0], w_ref[...], preferred_element_type=jnp.float32)
        o_ref[0] = m * dinv

    return pl.pallas_call(
        body,
        grid=(PERIODS, NBLK),
        in_specs=[
            pl.BlockSpec((1, BN, F_IN), lambda p, b: (p, b, 0)),
            pl.BlockSpec((F_IN, FW), lambda p, b: (0, 0)),
            pl.BlockSpec((NC, BN, 16), lambda p, b: (0, b, 0)),
        ],
        out_specs=pl.BlockSpec((1, BN, FW), lambda p, b: (p, b, 0)),
        out_shape=jax.ShapeDtypeStruct((PERIODS, NPAD, FW), jnp.float32),
    )(x_t, wcat, degpart)


def _tc_final(agg, degpart, probs, bz, bh, w1, b1, wout, bout):
    def body(a_ref, d_ref, pr_ref, bz_ref, bh_ref, w1_ref, b1_ref, wo_ref, bo_ref, o_ref):
        deg = d_ref[0, :, :1] + d_ref[1, :, :1] + 1.0
        dinv = lax.rsqrt(deg)
        acc = jnp.zeros((BN, HID), jnp.float32)
        for p in range(PERIODS):
            g = a_ref[p] * dinv
            zl = g[:, :HID] + bz_ref[...]
            hl = g[:, HID:] + bh_ref[...]
            hp = (1.0 - jax.nn.sigmoid(zl)) * jnp.tanh(hl)
            acc = acc + pr_ref[0, p] * hp
        h = jnp.maximum(acc, 0.0)
        h = jnp.maximum(
            jnp.dot(h, w1_ref[...], preferred_element_type=jnp.float32) + b1_ref[...],
            0.0,
        )
        o_ref[...] = (
            jnp.dot(h, wo_ref[...], preferred_element_type=jnp.float32) + bo_ref[...]
        )

    return pl.pallas_call(
        body,
        grid=(NBLK,),
        in_specs=[
            pl.BlockSpec((PERIODS, BN, FW), lambda b: (0, b, 0)),
            pl.BlockSpec((NC, BN, 16), lambda b: (0, b, 0)),
            pl.BlockSpec(memory_space=pltpu.SMEM),
            pl.BlockSpec((1, HID), lambda b: (0, 0)),
            pl.BlockSpec((1, HID), lambda b: (0, 0)),
            pl.BlockSpec((HID, HID), lambda b: (0, 0)),
            pl.BlockSpec((1, HID), lambda b: (0, 0)),
            pl.BlockSpec((HID, PERIODS), lambda b: (0, 0)),
            pl.BlockSpec((1, PERIODS), lambda b: (0, 0)),
        ],
        out_specs=pl.BlockSpec((BN, PERIODS), lambda b: (b, 0)),
        out_shape=jax.ShapeDtypeStruct((N, PERIODS), jnp.float32),
    )(agg, degpart, probs, bz, bh, w1, b1, wout, bout)


def kernel(x, edge_index, attention, Wcz, bcz, Wcr, bcr, Wch, bch, Wlz, blz, Wlr, blr, Wlh, blh, W1, b1, Wout, bout):
    src = edge_index[0]
    dst = edge_index[1]
    probs = jax.nn.softmax(attention).reshape(1, PERIODS)
    wcat = jnp.concatenate([Wcz @ Wlz[:HID], Wch @ Wlh[:HID]], axis=1)
    bz = (bcz @ Wlz[:HID] + blz).reshape(1, HID)
    bh = (bch @ Wlh[:HID] + blh).reshape(1, HID)
    x_t = jnp.transpose(x, (2, 0, 1))

    degpart = _sc_degree(dst.reshape(NC * NS, NBD, EB))
    mprime = _tc_prep(x_t, wcat, degpart)
    agg = _sc_aggregate(
        src.reshape(NS, NBE, EB), dst.reshape(NS, NBE, EB), mprime
    )
    return _tc_final(agg, degpart, probs, bz, bh, W1,
                     b1.reshape(1, HID), Wout, bout.reshape(1, PERIODS))


# speedup vs baseline: 61.3079x; 1.0998x over previous
import functools

import jax
import jax.numpy as jnp
from jax import lax
from jax.experimental import pallas as pl
from jax.experimental.pallas import tpu as pltpu
from jax.experimental.pallas import tpu_sc as plsc

N = 10000
NPAD = 10240
E = 320000
F_IN = 128
HID = 32
FW = 2 * HID
PERIODS = 12
NC = 2
NS = 16
EB = 125
GQ = 5
NBLK = 10
BN = N // NBLK

ROWS_PER_TILE = NPAD // NS
ZR = 80
NBD = E // (NC * NS * EB)
NBE = E // (NS * EB)

_SC_PARAMS = pltpu.CompilerParams(use_tc_tiling_on_sc=False)


def _sc_mesh():
    return plsc.VectorSubcoreMesh(
        core_axis_name="c", subcore_axis_name="s", num_cores=NC, num_subcores=NS
    )


def _zero_vmem_rows(ref, rows, width):
    @pl.loop(0, rows)
    def _(i):
        for q in range(width // 16):
            ref[i, pl.ds(q * 16, 16)] = jnp.zeros((16,), jnp.float32)


def _sc_degree(dst_r):
    @functools.partial(
        pl.kernel,
        out_type=jax.ShapeDtypeStruct((NC, NPAD, 16), jnp.float32),
        mesh=_sc_mesh(),
        compiler_params=_SC_PARAMS,
        scratch_types=[
            pltpu.VMEM((NBD, EB), jnp.int32),
            pltpu.VMEM((EB, 16), jnp.float32),
            pltpu.VMEM((ZR, 16), jnp.float32),
            pltpu.VMEM_SHARED((NPAD, 16), jnp.float32),
        ],
    )
    def k(dst_hbm, out_hbm, idx_v, ones_v, zer_v, acc_sh):
        c = lax.axis_index("c")
        s = lax.axis_index("s")
        wid = c * NS + s
        pltpu.sync_copy(dst_hbm.at[wid], idx_v)

        @pl.loop(0, EB)
        def _(i):
            ones_v[i, :] = jnp.ones((16,), jnp.float32)

        _zero_vmem_rows(zer_v, ZR, 16)
        row0 = s * ROWS_PER_TILE
        for z in range(ROWS_PER_TILE // ZR):
            pltpu.sync_copy(zer_v, acc_sh.at[pl.ds(row0 + z * ZR, ZR)])
        plsc.subcore_barrier()

        @pl.loop(0, NBD)
        def _(j):
            pltpu.sync_copy(ones_v, acc_sh.at[idx_v.at[j]], add=True)

        plsc.subcore_barrier()
        pltpu.sync_copy(
            acc_sh.at[pl.ds(row0, ROWS_PER_TILE)],
            out_hbm.at[c, pl.ds(row0, ROWS_PER_TILE)],
        )

    return k(dst_r)


def _sc_aggregate(src_r, dst_r, mprime):
    pp = PERIODS // NC
    ngroups = NBE // GQ

    @functools.partial(
        pl.kernel,
        out_type=jax.ShapeDtypeStruct((PERIODS, NPAD, FW), jnp.float32),
        mesh=_sc_mesh(),
        compiler_params=_SC_PARAMS,
        scratch_types=[
            pltpu.VMEM((NBE, EB), jnp.int32),
            pltpu.VMEM((NBE, EB), jnp.int32),
            pltpu.VMEM((GQ, EB, FW), jnp.float32),
            pltpu.VMEM_SHARED((NPAD, FW), jnp.float32),
            pltpu.SemaphoreType.DMA((GQ,)),
            pltpu.SemaphoreType.DMA((GQ,)),
        ],
    )
    def k(src_hbm, dst_hbm, mp_hbm, out_hbm, sidx, didx, gbuf, acc_sh, gsem, ssem):
        c = lax.axis_index("c")
        s = lax.axis_index("s")
        pltpu.sync_copy(src_hbm.at[s], sidx)
        pltpu.sync_copy(dst_hbm.at[s], didx)

        row0 = s * ROWS_PER_TILE
        for p_i in range(pp):
            p = c * pp + p_i
            tab = mp_hbm.at[p]
            pltpu.sync_copy(
                tab.at[pl.ds(row0, ROWS_PER_TILE)],
                acc_sh.at[pl.ds(row0, ROWS_PER_TILE)],
            )
            plsc.subcore_barrier()

            for b in range(GQ):
                pltpu.async_copy(tab.at[sidx.at[b]], gbuf.at[b], gsem.at[b])

            @pl.loop(0, ngroups)
            def _(g):
                base = g * GQ
                for b in range(GQ):
                    pltpu.make_async_copy(
                        tab.at[sidx.at[base + b]], gbuf.at[b], gsem.at[b]
                    ).wait()
                    pltpu.async_copy(
                        gbuf.at[b], acc_sh.at[didx.at[base + b]], ssem.at[b],
                        add=True,
                    )

                @pl.when(g < ngroups - 1)
                def _():
                    nbase = base + GQ
                    for b in range(GQ):
                        pltpu.make_async_copy(
                            gbuf.at[b], acc_sh.at[didx.at[base + b]], ssem.at[b]
                        ).wait()
                        pltpu.async_copy(
                            tab.at[sidx.at[nbase + b]], gbuf.at[b], gsem.at[b]
                        )

            for b in range(GQ):
                pltpu.make_async_copy(
                    gbuf.at[b], acc_sh.at[didx.at[b]], ssem.at[b]
                ).wait()
            plsc.subcore_barrier()
            pltpu.sync_copy(
                acc_sh.at[pl.ds(row0, ROWS_PER_TILE)],
                out_hbm.at[p, pl.ds(row0, ROWS_PER_TILE)],
            )
            plsc.subcore_barrier()

    return k(src_r, dst_r, mprime)


def _tc_prep(x_t, wcat, degpart):
    def body(x_ref, w_ref, d_ref, o_ref):
        deg = d_ref[0, :, :1] + d_ref[1, :, :1] + 1.0
        dinv = lax.rsqrt(deg)
        w = w_ref[...]
        for p in range(PERIODS):
            o_ref[p] = (
                jnp.dot(x_ref[p], w, preferred_element_type=jnp.float32) * dinv
            )

    return pl.pallas_call(
        body,
        grid=(NBLK,),
        in_specs=[
            pl.BlockSpec((PERIODS, BN, F_IN), lambda b: (0, b, 0)),
            pl.BlockSpec((F_IN, FW), lambda b: (0, 0)),
            pl.BlockSpec((NC, BN, 16), lambda b: (0, b, 0)),
        ],
        out_specs=pl.BlockSpec((PERIODS, BN, FW), lambda b: (0, b, 0)),
        out_shape=jax.ShapeDtypeStruct((PERIODS, NPAD, FW), jnp.float32),
    )(x_t, wcat, degpart)


def _tc_final(agg, degpart, probs, bz, bh, w1, b1, wout, bout):
    def body(a_ref, d_ref, pr_ref, bz_ref, bh_ref, w1_ref, b1_ref, wo_ref, bo_ref, o_ref):
        deg = d_ref[0, :, :1] + d_ref[1, :, :1] + 1.0
        dinv = lax.rsqrt(deg)
        acc = jnp.zeros((BN, HID), jnp.float32)
        for p in range(PERIODS):
            g = a_ref[p] * dinv
            zl = g[:, :HID] + bz_ref[...]
            hl = g[:, HID:] + bh_ref[...]
            hp = (1.0 - jax.nn.sigmoid(zl)) * jnp.tanh(hl)
            acc = acc + pr_ref[0, p] * hp
        h = jnp.maximum(acc, 0.0)
        h = jnp.maximum(
            jnp.dot(h, w1_ref[...], preferred_element_type=jnp.float32) + b1_ref[...],
            0.0,
        )
        o_ref[...] = (
            jnp.dot(h, wo_ref[...], preferred_element_type=jnp.float32) + bo_ref[...]
        )

    return pl.pallas_call(
        body,
        grid=(NBLK,),
        in_specs=[
            pl.BlockSpec((PERIODS, BN, FW), lambda b: (0, b, 0)),
            pl.BlockSpec((NC, BN, 16), lambda b: (0, b, 0)),
            pl.BlockSpec(memory_space=pltpu.SMEM),
            pl.BlockSpec((1, HID), lambda b: (0, 0)),
            pl.BlockSpec((1, HID), lambda b: (0, 0)),
            pl.BlockSpec((HID, HID), lambda b: (0, 0)),
            pl.BlockSpec((1, HID), lambda b: (0, 0)),
            pl.BlockSpec((HID, PERIODS), lambda b: (0, 0)),
            pl.BlockSpec((1, PERIODS), lambda b: (0, 0)),
        ],
        out_specs=pl.BlockSpec((BN, PERIODS), lambda b: (b, 0)),
        out_shape=jax.ShapeDtypeStruct((N, PERIODS), jnp.float32),
    )(agg, degpart, probs, bz, bh, w1, b1, wout, bout)


def kernel(x, edge_index, attention, Wcz, bcz, Wcr, bcr, Wch, bch, Wlz, blz, Wlr, blr, Wlh, blh, W1, b1, Wout, bout):
    src = edge_index[0]
    dst = edge_index[1]
    probs = jax.nn.softmax(attention).reshape(1, PERIODS)
    wcat = jnp.concatenate([Wcz @ Wlz[:HID], Wch @ Wlh[:HID]], axis=1)
    bz = (bcz @ Wlz[:HID] + blz).reshape(1, HID)
    bh = (bch @ Wlh[:HID] + blh).reshape(1, HID)
    x_t = jnp.transpose(x, (2, 0, 1))

    degpart = _sc_degree(dst.reshape(NC * NS, NBD, EB))
    mprime = _tc_prep(x_t, wcat, degpart)
    agg = _sc_aggregate(
        src.reshape(NS, NBE, EB), dst.reshape(NS, NBE, EB), mprime
    )
    return _tc_final(agg, degpart, probs, bz, bh, W1,
                     b1.reshape(1, HID), Wout, bout.reshape(1, PERIODS))


# speedup vs baseline: 61.4484x; 1.0023x over previous
import functools

import jax
import jax.numpy as jnp
from jax import lax
from jax.experimental import pallas as pl
from jax.experimental.pallas import tpu as pltpu
from jax.experimental.pallas import tpu_sc as plsc

N = 10000
NPAD = 10240
E = 320000
F_IN = 128
HID = 32
FW = 2 * HID
PERIODS = 12
NC = 2
NS = 16
EB = 125
GQ = 5
NBLK = 5
BN = N // NBLK

ROWS_PER_TILE = NPAD // NS
ZR = 80
NBD = E // (NC * NS * EB)
NBE = E // (NS * EB)

_SC_PARAMS = pltpu.CompilerParams(use_tc_tiling_on_sc=False)


def _sc_mesh():
    return plsc.VectorSubcoreMesh(
        core_axis_name="c", subcore_axis_name="s", num_cores=NC, num_subcores=NS
    )


def _zero_vmem_rows(ref, rows, width):
    @pl.loop(0, rows)
    def _(i):
        for q in range(width // 16):
            ref[i, pl.ds(q * 16, 16)] = jnp.zeros((16,), jnp.float32)


def _sc_degree(dst_r):
    @functools.partial(
        pl.kernel,
        out_type=jax.ShapeDtypeStruct((NC, NPAD, 16), jnp.float32),
        mesh=_sc_mesh(),
        compiler_params=_SC_PARAMS,
        scratch_types=[
            pltpu.VMEM((NBD, EB), jnp.int32),
            pltpu.VMEM((EB, 16), jnp.float32),
            pltpu.VMEM((ZR, 16), jnp.float32),
            pltpu.VMEM_SHARED((NPAD, 16), jnp.float32),
        ],
    )
    def k(dst_hbm, out_hbm, idx_v, ones_v, zer_v, acc_sh):
        c = lax.axis_index("c")
        s = lax.axis_index("s")
        wid = c * NS + s
        pltpu.sync_copy(dst_hbm.at[wid], idx_v)

        @pl.loop(0, EB)
        def _(i):
            ones_v[i, :] = jnp.ones((16,), jnp.float32)

        _zero_vmem_rows(zer_v, ZR, 16)
        row0 = s * ROWS_PER_TILE
        for z in range(ROWS_PER_TILE // ZR):
            pltpu.sync_copy(zer_v, acc_sh.at[pl.ds(row0 + z * ZR, ZR)])
        plsc.subcore_barrier()

        @pl.loop(0, NBD)
        def _(j):
            pltpu.sync_copy(ones_v, acc_sh.at[idx_v.at[j]], add=True)

        plsc.subcore_barrier()
        pltpu.sync_copy(
            acc_sh.at[pl.ds(row0, ROWS_PER_TILE)],
            out_hbm.at[c, pl.ds(row0, ROWS_PER_TILE)],
        )

    return k(dst_r)


def _sc_aggregate(src_r, dst_r, mprime):
    pp = PERIODS // NC
    ngroups = NBE // GQ

    @functools.partial(
        pl.kernel,
        out_type=jax.ShapeDtypeStruct((PERIODS, NPAD, FW), jnp.float32),
        mesh=_sc_mesh(),
        compiler_params=_SC_PARAMS,
        scratch_types=[
            pltpu.VMEM((NBE, EB), jnp.int32),
            pltpu.VMEM((NBE, EB), jnp.int32),
            pltpu.VMEM((GQ, EB, FW), jnp.float32),
            pltpu.VMEM_SHARED((NPAD, FW), jnp.float32),
            pltpu.SemaphoreType.DMA((GQ,)),
            pltpu.SemaphoreType.DMA((GQ,)),
        ],
    )
    def k(src_hbm, dst_hbm, mp_hbm, out_hbm, sidx, didx, gbuf, acc_sh, gsem, ssem):
        c = lax.axis_index("c")
        s = lax.axis_index("s")
        pltpu.sync_copy(src_hbm.at[s], sidx)
        pltpu.sync_copy(dst_hbm.at[s], didx)

        row0 = s * ROWS_PER_TILE
        for p_i in range(pp):
            p = c * pp + p_i
            tab = mp_hbm.at[p]
            pltpu.sync_copy(
                tab.at[pl.ds(row0, ROWS_PER_TILE)],
                acc_sh.at[pl.ds(row0, ROWS_PER_TILE)],
            )
            plsc.subcore_barrier()

            for b in range(GQ):
                pltpu.async_copy(tab.at[sidx.at[b]], gbuf.at[b], gsem.at[b])

            @pl.loop(0, ngroups)
            def _(g):
                base = g * GQ
                for b in range(GQ):
                    pltpu.make_async_copy(
                        tab.at[sidx.at[base + b]], gbuf.at[b], gsem.at[b]
                    ).wait()
                    pltpu.async_copy(
                        gbuf.at[b], acc_sh.at[didx.at[base + b]], ssem.at[b],
                        add=True,
                    )

                @pl.when(g < ngroups - 1)
                def _():
                    nbase = base + GQ
                    for b in range(GQ):
                        pltpu.make_async_copy(
                            gbuf.at[b], acc_sh.at[didx.at[base + b]], ssem.at[b]
                        ).wait()
                        pltpu.async_copy(
                            tab.at[sidx.at[nbase + b]], gbuf.at[b], gsem.at[b]
                        )

            for b in range(GQ):
                pltpu.make_async_copy(
                    gbuf.at[b], acc_sh.at[didx.at[b]], ssem.at[b]
                ).wait()
            plsc.subcore_barrier()
            pltpu.sync_copy(
                acc_sh.at[pl.ds(row0, ROWS_PER_TILE)],
                out_hbm.at[p, pl.ds(row0, ROWS_PER_TILE)],
            )
            plsc.subcore_barrier()

    return k(src_r, dst_r, mprime)


def _tc_prep(x_t, wcat, degpart):
    def body(x_ref, w_ref, d_ref, o_ref):
        deg = d_ref[0, :, :1] + d_ref[1, :, :1] + 1.0
        dinv = lax.rsqrt(deg)
        w = w_ref[...]
        for p in range(PERIODS):
            m = jnp.dot(x_ref[p], w, preferred_element_type=jnp.float32) * dinv
            o_ref[p] = m

    return pl.pallas_call(
        body,
        grid=(NBLK,),
        in_specs=[
            pl.BlockSpec((PERIODS, BN, F_IN), lambda b: (0, b, 0)),
            pl.BlockSpec((F_IN, FW), lambda b: (0, 0)),
            pl.BlockSpec((NC, BN, 16), lambda b: (0, b, 0)),
        ],
        out_specs=pl.BlockSpec((PERIODS, BN, FW), lambda b: (0, b, 0)),
        out_shape=jax.ShapeDtypeStruct((PERIODS, NPAD, FW), jnp.float32),
    )(x_t, wcat, degpart)


def _tc_final(agg, degpart, probs, bz, bh, w1, b1, wout, bout):
    def body(a_ref, d_ref, pr_ref, bz_ref, bh_ref, w1_ref, b1_ref, wo_ref, bo_ref, o_ref):
        deg = d_ref[0, :, :1] + d_ref[1, :, :1] + 1.0
        dinv = lax.rsqrt(deg)
        acc = jnp.zeros((BN, HID), jnp.float32)
        for p in range(PERIODS):
            g = a_ref[p] * dinv
            zl = g[:, :HID] + bz_ref[...]
            hl = g[:, HID:] + bh_ref[...]
            hp = (1.0 - jax.nn.sigmoid(zl)) * jnp.tanh(hl)
            acc = acc + pr_ref[0, p] * hp
        h = jnp.maximum(acc, 0.0)
        h = jnp.maximum(
            jnp.dot(h, w1_ref[...], preferred_element_type=jnp.float32) + b1_ref[...],
            0.0,
        )
        o_ref[...] = (
            jnp.dot(h, wo_ref[...], preferred_element_type=jnp.float32) + bo_ref[...]
        )

    return pl.pallas_call(
        body,
        grid=(NBLK,),
        in_specs=[
            pl.BlockSpec((PERIODS, BN, FW), lambda b: (0, b, 0)),
            pl.BlockSpec((NC, BN, 16), lambda b: (0, b, 0)),
            pl.BlockSpec(memory_space=pltpu.SMEM),
            pl.BlockSpec((1, HID), lambda b: (0, 0)),
            pl.BlockSpec((1, HID), lambda b: (0, 0)),
            pl.BlockSpec((HID, HID), lambda b: (0, 0)),
            pl.BlockSpec((1, HID), lambda b: (0, 0)),
            pl.BlockSpec((HID, PERIODS), lambda b: (0, 0)),
            pl.BlockSpec((1, PERIODS), lambda b: (0, 0)),
        ],
        out_specs=pl.BlockSpec((BN, PERIODS), lambda b: (b, 0)),
        out_shape=jax.ShapeDtypeStruct((N, PERIODS), jnp.float32),
    )(agg, degpart, probs, bz, bh, w1, b1, wout, bout)


def kernel(x, edge_index, attention, Wcz, bcz, Wcr, bcr, Wch, bch, Wlz, blz, Wlr, blr, Wlh, blh, W1, b1, Wout, bout):
    src = edge_index[0]
    dst = edge_index[1]
    probs = jax.nn.softmax(attention).reshape(1, PERIODS)
    wcat = jnp.concatenate([Wcz @ Wlz[:HID], Wch @ Wlh[:HID]], axis=1)
    bz = (bcz @ Wlz[:HID] + blz).reshape(1, HID)
    bh = (bch @ Wlh[:HID] + blh).reshape(1, HID)
    x_t = jnp.transpose(x, (2, 0, 1))

    degpart = _sc_degree(dst.reshape(NC * NS, NBD, EB))
    mprime = _tc_prep(x_t, wcat, degpart)
    agg = _sc_aggregate(
        src.reshape(NS, NBE, EB), dst.reshape(NS, NBE, EB), mprime
    )
    return _tc_final(agg, degpart, probs,
                     bz, bh, W1, b1.reshape(1, HID), Wout,
                     bout.reshape(1, PERIODS))
